# baseline (speedup 1.0000x reference)
import jax
import jax.numpy as jnp
from jax import lax
from jax.experimental import pallas as pl
from jax.experimental.pallas import tpu as pltpu
from jax.experimental.pallas import tpu_sc as plsc

_GS = 8
_NZ = 4
_DELTA = 127.0
_MAGIC8 = 100663296.0
_INV1016 = 1.0 / (8.0 * _DELTA)

_NC, _NS = 2, 16
_NW = _NC * _NS


def _maxabs_body(x_ref, o_ref):
    i = pl.program_id(0)
    m = jnp.max(jnp.abs(x_ref[...]))

    @pl.when(i == 0)
    def _init():
        o_ref[0, 0] = m

    @pl.when(i > 0)
    def _acc():
        o_ref[0, 0] = jnp.maximum(o_ref[0, 0], m)


def _ce(a, b):
    return jnp.minimum(a, b), jnp.maximum(a, b)


def _sc_quant_unit(ibuf, obuf, sv8, ubase):
    iota8 = lax.iota(jnp.int32, 16) * _GS + ubase
    x = []
    for k in range(_GS):
        x.append(plsc.load_gather(ibuf, [iota8 + k]))
    key = []
    for k in range(_GS):
        ax = jnp.abs(x[k])
        e = jnp.exp(ax * -2.0)
        m8 = sv8 * (1.0 - e) / (1.0 + e)
        key.append(((m8 + _MAGIC8) - _MAGIC8) + float(k))
    x0, x1, x2, x3, y0, y1, y2, y3 = key
    x0, x1 = _ce(x0, x1); x2, x3 = _ce(x2, x3)
    x0, x2 = _ce(x0, x2); x1, x3 = _ce(x1, x3); x1, x2 = _ce(x1, x2)
    y0, y1 = _ce(y0, y1); y2, y3 = _ce(y2, y3)
    y0, y2 = _ce(y0, y2); y1, y3 = _ce(y1, y3); y1, y2 = _ce(y1, y2)
    t = jnp.minimum(
        jnp.minimum(jnp.maximum(x0, y3), jnp.maximum(x1, y2)),
        jnp.minimum(jnp.maximum(x2, y1), jnp.maximum(x3, y0)),
    )
    for k in range(_GS):
        sc = jnp.where(x[k] < 0.0, -_INV1016, _INV1016)
        out = jnp.where(key[k] >= t, key[k] - float(k), 0.0) * sc
        plsc.store_scatter(obuf, [iota8 + k], out)


def _sc_quant_kernel(chunk, units, nchunks):
    npairs = nchunks // 2

    def body(x_hbm, s_hbm, o_hbm, ib0, ib1, ob0, ob1, svec,
             is0, is1, os0, os1):
        wid = lax.axis_index("s") * _NC + lax.axis_index("c")
        ibufs, obufs = (ib0, ib1), (ob0, ob1)
        isems, osems = (is0, is1), (os0, os1)
        pltpu.sync_copy(s_hbm, svec)
        sv = svec[...]
        base = wid * (nchunks * chunk)

        def in_copy(g, b):
            return pltpu.make_async_copy(
                x_hbm.at[pl.ds(base + g * chunk, chunk)], ibufs[b], isems[b])

        def out_copy(g, b):
            return pltpu.make_async_copy(
                obufs[b], o_hbm.at[pl.ds(base + g * chunk, chunk)], osems[b])

        for b in range(2):
            in_copy(b, b).start()

        def do_pair(p, _):
            for b in range(2):
                g = p * 2 + b
                in_copy(g, b).wait()

                @pl.when(p >= 1)
                def _drain():
                    out_copy(g - 2, b).wait()

                @plsc.parallel_loop(0, units, 1, unroll=2)
                def do_unit(u):
                    _sc_quant_unit(ibufs[b], obufs[b], sv, u * 128)
                out_copy(g, b).start()

                @pl.when(p < npairs - 1)
                def _prefetch():
                    in_copy(g + 2, b).start()

            return 0

        lax.fori_loop(0, npairs, do_pair, 0)
        for b in range(2):
            out_copy(nchunks - 2 + b, b).wait()

    return body


def kernel(x):
    out_c, in_c, h, w = x.shape
    hw = h * w
    xt = jnp.transpose(x, (2, 3, 0, 1)).reshape(hw, out_c, in_c)
    n = hw * out_c * in_c
    xf = xt.reshape(n)

    b1 = 2
    maxabs = pl.pallas_call(
        _maxabs_body,
        grid=(hw // b1,),
        in_specs=[pl.BlockSpec((b1, out_c, in_c), lambda i: (i, 0, 0))],
        out_specs=pl.BlockSpec(memory_space=pltpu.SMEM),
        out_shape=jax.ShapeDtypeStruct((1, 1), jnp.float32),
    )(xt)
    scale = jnp.broadcast_to(8.0 * _DELTA / jnp.tanh(maxabs[0, 0]), (16,))

    chunk = 18816
    units = chunk // 128
    assert n % (chunk * _NW) == 0
    nchunks = n // (chunk * _NW)
    assert nchunks % 2 == 0
    mesh = plsc.VectorSubcoreMesh(core_axis_name="c", subcore_axis_name="s")
    out = pl.kernel(
        _sc_quant_kernel(chunk, units, nchunks),
        mesh=mesh,
        out_type=jax.ShapeDtypeStruct((n,), jnp.float32),
        compiler_params=pltpu.CompilerParams(needs_layout_passes=False),
        scratch_types=[
            pltpu.VMEM((chunk,), jnp.float32),
            pltpu.VMEM((chunk,), jnp.float32),
            pltpu.VMEM((chunk,), jnp.float32),
            pltpu.VMEM((chunk,), jnp.float32),
            pltpu.VMEM((16,), jnp.float32),
            pltpu.SemaphoreType.DMA,
            pltpu.SemaphoreType.DMA,
            pltpu.SemaphoreType.DMA,
            pltpu.SemaphoreType.DMA,
        ],
    )(xf, scale)
    return jnp.transpose(out.reshape(h, w, out_c, in_c), (2, 3, 0, 1))


# speedup vs baseline: 1.1008x; 1.1008x over previous
import jax
import jax.numpy as jnp
from jax import lax
from jax.experimental import pallas as pl
from jax.experimental.pallas import tpu as pltpu
from jax.experimental.pallas import tpu_sc as plsc

_GS = 8
_NZ = 4
_DELTA = 127.0
_MAGIC8 = 100663296.0
_INV1016 = 1.0 / (8.0 * _DELTA)

_NC, _NS = 2, 16
_NW = _NC * _NS


def _maxabs_body(x_ref, o_ref):
    i = pl.program_id(0)
    m = jnp.max(jnp.abs(x_ref[...]))

    @pl.when(i == 0)
    def _init():
        o_ref[0, 0] = m

    @pl.when(i > 0)
    def _acc():
        o_ref[0, 0] = jnp.maximum(o_ref[0, 0], m)


def _ce(a, b):
    return jnp.minimum(a, b), jnp.maximum(a, b)


def _sc_quant_unit(ibuf, obuf, sv8, ubase):
    iota8 = lax.iota(jnp.int32, 16) * _GS + ubase
    x = []
    for k in range(_GS):
        x.append(plsc.load_gather(ibuf, [iota8 + k]))
    key = []
    for k in range(_GS):
        ax = jnp.abs(x[k])
        e = jnp.exp(ax * -2.0)
        m8 = sv8 * (1.0 - e) / (1.0 + e)
        key.append(((m8 + _MAGIC8) - _MAGIC8) + float(k))
    x0, x1, x2, x3, y0, y1, y2, y3 = key
    x0, x1 = _ce(x0, x1); x2, x3 = _ce(x2, x3)
    x0, x2 = _ce(x0, x2); x1, x3 = _ce(x1, x3); x1, x2 = _ce(x1, x2)
    y0, y1 = _ce(y0, y1); y2, y3 = _ce(y2, y3)
    y0, y2 = _ce(y0, y2); y1, y3 = _ce(y1, y3); y1, y2 = _ce(y1, y2)
    t = jnp.minimum(
        jnp.minimum(jnp.maximum(x0, y3), jnp.maximum(x1, y2)),
        jnp.minimum(jnp.maximum(x2, y1), jnp.maximum(x3, y0)),
    )
    for k in range(_GS):
        sc = jnp.where(x[k] < 0.0, -_INV1016, _INV1016)
        out = jnp.where(key[k] >= t, key[k] - float(k), 0.0) * sc
        plsc.store_scatter(obuf, [iota8 + k], out)


def _sc_quant_kernel(chunk, units, nchunks):
    npairs = nchunks // 2

    def body(x_hbm, s_hbm, o_hbm, ib0, ib1, ob0, ob1, svec,
             is0, is1, os0, os1):
        wid = lax.axis_index("s") * _NC + lax.axis_index("c")
        ibufs, obufs = (ib0, ib1), (ob0, ob1)
        isems, osems = (is0, is1), (os0, os1)
        pltpu.sync_copy(s_hbm, svec)
        sv = svec[...]
        base = wid * (nchunks * chunk)

        def in_copy(g, b):
            return pltpu.make_async_copy(
                x_hbm.at[pl.ds(base + g * chunk, chunk)], ibufs[b], isems[b])

        def out_copy(g, b):
            return pltpu.make_async_copy(
                obufs[b], o_hbm.at[pl.ds(base + g * chunk, chunk)], osems[b])

        for b in range(2):
            in_copy(b, b).start()

        def do_pair(p, _):
            for b in range(2):
                g = p * 2 + b
                in_copy(g, b).wait()

                @pl.when(p >= 1)
                def _drain():
                    out_copy(g - 2, b).wait()

                @plsc.parallel_loop(0, units, 1, unroll=1)
                def do_unit(u):
                    _sc_quant_unit(ibufs[b], obufs[b], sv, u * 128)
                out_copy(g, b).start()

                @pl.when(p < npairs - 1)
                def _prefetch():
                    in_copy(g + 2, b).start()

            return 0

        lax.fori_loop(0, npairs, do_pair, 0)
        for b in range(2):
            out_copy(nchunks - 2 + b, b).wait()

    return body


def kernel(x):
    out_c, in_c, h, w = x.shape
    hw = h * w
    xt = jnp.transpose(x, (2, 3, 0, 1)).reshape(hw, out_c, in_c)
    n = hw * out_c * in_c
    xf = xt.reshape(n)

    b1 = 2
    maxabs = pl.pallas_call(
        _maxabs_body,
        grid=(hw // b1,),
        in_specs=[pl.BlockSpec((b1, out_c, in_c), lambda i: (i, 0, 0))],
        out_specs=pl.BlockSpec(memory_space=pltpu.SMEM),
        out_shape=jax.ShapeDtypeStruct((1, 1), jnp.float32),
    )(xt)
    scale = jnp.broadcast_to(8.0 * _DELTA / jnp.tanh(maxabs[0, 0]), (16,))

    chunk = 18816
    units = chunk // 128
    assert n % (chunk * _NW) == 0
    nchunks = n // (chunk * _NW)
    assert nchunks % 2 == 0
    mesh = plsc.VectorSubcoreMesh(core_axis_name="c", subcore_axis_name="s")
    out = pl.kernel(
        _sc_quant_kernel(chunk, units, nchunks),
        mesh=mesh,
        out_type=jax.ShapeDtypeStruct((n,), jnp.float32),
        compiler_params=pltpu.CompilerParams(needs_layout_passes=False),
        scratch_types=[
            pltpu.VMEM((chunk,), jnp.float32),
            pltpu.VMEM((chunk,), jnp.float32),
            pltpu.VMEM((chunk,), jnp.float32),
            pltpu.VMEM((chunk,), jnp.float32),
            pltpu.VMEM((16,), jnp.float32),
            pltpu.SemaphoreType.DMA,
            pltpu.SemaphoreType.DMA,
            pltpu.SemaphoreType.DMA,
            pltpu.SemaphoreType.DMA,
        ],
    )(xf, scale)
    return jnp.transpose(out.reshape(h, w, out_c, in_c), (2, 3, 0, 1))


# speedup vs baseline: 1.1363x; 1.0323x over previous
import jax
import jax.numpy as jnp
from jax import lax
from jax.experimental import pallas as pl
from jax.experimental.pallas import tpu as pltpu
from jax.experimental.pallas import tpu_sc as plsc

_GS = 8
_NZ = 4
_DELTA = 127.0
_MAGIC8 = 100663296.0
_INV1016 = 1.0 / (8.0 * _DELTA)

_NC, _NS = 2, 16
_NW = _NC * _NS


def _maxabs_body(x_ref, o_ref):
    i = pl.program_id(0)
    m = jnp.max(jnp.abs(x_ref[...]))

    @pl.when(i == 0)
    def _init():
        o_ref[0, 0] = m

    @pl.when(i > 0)
    def _acc():
        o_ref[0, 0] = jnp.maximum(o_ref[0, 0], m)


def _ce(a, b):
    return jnp.minimum(a, b), jnp.maximum(a, b)


def _sc_quant_unit(ibuf, obuf, sv8, sv16, ubase):
    iota8 = lax.iota(jnp.int32, 16) * _GS + ubase
    x = []
    for k in range(_GS):
        x.append(plsc.load_gather(ibuf, [iota8 + k]))
    key = []
    for k in range(_GS):
        ax = jnp.abs(x[k])
        e = jnp.exp(ax * -2.0)
        m8 = sv16 / (1.0 + e) - sv8
        key.append(((m8 + _MAGIC8) - _MAGIC8) + float(k))
    x0, x1, x2, x3, y0, y1, y2, y3 = key
    x0, x1 = _ce(x0, x1); x2, x3 = _ce(x2, x3)
    x0, x2 = _ce(x0, x2); x1, x3 = _ce(x1, x3); x1, x2 = _ce(x1, x2)
    y0, y1 = _ce(y0, y1); y2, y3 = _ce(y2, y3)
    y0, y2 = _ce(y0, y2); y1, y3 = _ce(y1, y3); y1, y2 = _ce(y1, y2)
    t = jnp.minimum(
        jnp.minimum(jnp.maximum(x0, y3), jnp.maximum(x1, y2)),
        jnp.minimum(jnp.maximum(x2, y1), jnp.maximum(x3, y0)),
    )
    for k in range(_GS):
        sc = jnp.where(x[k] < 0.0, -_INV1016, _INV1016)
        out = jnp.where(key[k] >= t, key[k] - float(k), 0.0) * sc
        plsc.store_scatter(obuf, [iota8 + k], out)


def _sc_quant_kernel(chunk, units, nchunks):
    npairs = nchunks // 2

    def body(x_hbm, s_hbm, o_hbm, ib0, ib1, ob0, ob1, svec,
             is0, is1, os0, os1):
        wid = lax.axis_index("s") * _NC + lax.axis_index("c")
        ibufs, obufs = (ib0, ib1), (ob0, ob1)
        isems, osems = (is0, is1), (os0, os1)
        pltpu.sync_copy(s_hbm, svec)
        sv = svec[...]
        sv16 = sv + sv
        base = wid * (nchunks * chunk)

        def in_copy(g, b):
            return pltpu.make_async_copy(
                x_hbm.at[pl.ds(base + g * chunk, chunk)], ibufs[b], isems[b])

        def out_copy(g, b):
            return pltpu.make_async_copy(
                obufs[b], o_hbm.at[pl.ds(base + g * chunk, chunk)], osems[b])

        for b in range(2):
            in_copy(b, b).start()

        def do_pair(p, _):
            for b in range(2):
                g = p * 2 + b
                in_copy(g, b).wait()

                @pl.when(p >= 1)
                def _drain():
                    out_copy(g - 2, b).wait()

                @plsc.parallel_loop(0, units, 1, unroll=1)
                def do_unit(u):
                    _sc_quant_unit(ibufs[b], obufs[b], sv, sv16, u * 128)
                out_copy(g, b).start()

                @pl.when(p < npairs - 1)
                def _prefetch():
                    in_copy(g + 2, b).start()

            return 0

        lax.fori_loop(0, npairs, do_pair, 0)
        for b in range(2):
            out_copy(nchunks - 2 + b, b).wait()

    return body


def kernel(x):
    out_c, in_c, h, w = x.shape
    hw = h * w
    xt = jnp.transpose(x, (2, 3, 0, 1)).reshape(hw, out_c, in_c)
    n = hw * out_c * in_c
    xf = xt.reshape(n)

    b1 = 2
    maxabs = pl.pallas_call(
        _maxabs_body,
        grid=(hw // b1,),
        in_specs=[pl.BlockSpec((b1, out_c, in_c), lambda i: (i, 0, 0))],
        out_specs=pl.BlockSpec(memory_space=pltpu.SMEM),
        out_shape=jax.ShapeDtypeStruct((1, 1), jnp.float32),
    )(xt)
    scale = jnp.broadcast_to(8.0 * _DELTA / jnp.tanh(maxabs[0, 0]), (16,))

    chunk = 28224
    units = chunk // 128
    assert n % (chunk * _NW) == 0
    nchunks = n // (chunk * _NW)
    assert nchunks % 2 == 0
    mesh = plsc.VectorSubcoreMesh(core_axis_name="c", subcore_axis_name="s")
    out = pl.kernel(
        _sc_quant_kernel(chunk, units, nchunks),
        mesh=mesh,
        out_type=jax.ShapeDtypeStruct((n,), jnp.float32),
        compiler_params=pltpu.CompilerParams(needs_layout_passes=False),
        scratch_types=[
            pltpu.VMEM((chunk,), jnp.float32),
            pltpu.VMEM((chunk,), jnp.float32),
            pltpu.VMEM((chunk,), jnp.float32),
            pltpu.VMEM((chunk,), jnp.float32),
            pltpu.VMEM((16,), jnp.float32),
            pltpu.SemaphoreType.DMA,
            pltpu.SemaphoreType.DMA,
            pltpu.SemaphoreType.DMA,
            pltpu.SemaphoreType.DMA,
        ],
    )(xf, scale)
    return jnp.transpose(out.reshape(h, w, out_c, in_c), (2, 3, 0, 1))


# speedup vs baseline: 1.6115x; 1.4181x over previous
import jax
import jax.numpy as jnp
from jax import lax
from jax.experimental import pallas as pl
from jax.experimental.pallas import tpu as pltpu
from jax.experimental.pallas import tpu_sc as plsc

_GS = 8
_NZ = 4
_DELTA = 127.0
_MAGIC8 = 100663296.0
_INV1016 = 1.0 / (8.0 * _DELTA)

_NC, _NS = 2, 16
_NW = _NC * _NS


def _maxabs_body(x_ref, o_ref):
    i = pl.program_id(0)
    m = jnp.max(jnp.abs(x_ref[...]))

    @pl.when(i == 0)
    def _init():
        o_ref[0, 0] = m

    @pl.when(i > 0)
    def _acc():
        o_ref[0, 0] = jnp.maximum(o_ref[0, 0], m)


def _ce(a, b):
    return jnp.minimum(a, b), jnp.maximum(a, b)


def _sc_quant_unit(ibuf, obuf, sv8, sv16, rvec, cbase):
    iota8 = lax.iota(jnp.int32, 16) * _GS + cbase
    x = []
    for k in range(_GS):
        x.append(plsc.load_gather(ibuf, [rvec, iota8 + k]))
    key = []
    for k in range(_GS):
        ax = jnp.abs(x[k])
        e = jnp.exp(ax * -2.0)
        m8 = sv16 / (1.0 + e) - sv8
        key.append(((m8 + _MAGIC8) - _MAGIC8) + float(k))
    x0, x1, x2, x3, y0, y1, y2, y3 = key
    x0, x1 = _ce(x0, x1); x2, x3 = _ce(x2, x3)
    x0, x2 = _ce(x0, x2); x1, x3 = _ce(x1, x3); x1, x2 = _ce(x1, x2)
    y0, y1 = _ce(y0, y1); y2, y3 = _ce(y2, y3)
    y0, y2 = _ce(y0, y2); y1, y3 = _ce(y1, y3); y1, y2 = _ce(y1, y2)
    t = jnp.minimum(
        jnp.minimum(jnp.maximum(x0, y3), jnp.maximum(x1, y2)),
        jnp.minimum(jnp.maximum(x2, y1), jnp.maximum(x3, y0)),
    )
    for k in range(_GS):
        sc = jnp.where(x[k] < 0.0, -_INV1016, _INV1016)
        out = jnp.where(key[k] >= t, key[k] - float(k), 0.0) * sc
        plsc.store_scatter(obuf, [rvec, iota8 + k], out)


def _sc_quant_kernel(crows, ncols, nchunks):
    npairs = nchunks // 2
    cunits = ncols // 128

    def body(x_hbm, s_hbm, o_hbm, ib0, ib1, ob0, ob1, svec,
             is0, is1, os0, os1):
        wid = lax.axis_index("s") * _NC + lax.axis_index("c")
        ibufs, obufs = (ib0, ib1), (ob0, ob1)
        isems, osems = (is0, is1), (os0, os1)
        pltpu.sync_copy(s_hbm, svec)
        sv = svec[...]
        sv16 = sv + sv
        base = wid * nchunks
        zero16 = jnp.zeros((16,), jnp.int32)

        def in_copy(g, b):
            return pltpu.make_async_copy(
                x_hbm.at[pl.ds((base + g) * crows, crows)], ibufs[b],
                isems[b])

        def out_copy(g, b):
            return pltpu.make_async_copy(
                obufs[b], o_hbm.at[pl.ds((base + g) * crows, crows)],
                osems[b])

        for b in range(2):
            in_copy(b, b).start()

        def do_pair(p, _):
            for b in range(2):
                g = p * 2 + b
                in_copy(g, b).wait()

                @pl.when(p >= 1)
                def _drain():
                    out_copy(g - 2, b).wait()

                @plsc.parallel_loop(0, crows, 1, unroll=1)
                def do_row(r):
                    rvec = zero16 + r
                    for c in range(cunits):
                        _sc_quant_unit(ibufs[b], obufs[b], sv, sv16,
                                       rvec, c * 128)
                out_copy(g, b).start()

                @pl.when(p < npairs - 1)
                def _prefetch():
                    in_copy(g + 2, b).start()

            return 0

        lax.fori_loop(0, npairs, do_pair, 0)
        for b in range(2):
            out_copy(nchunks - 2 + b, b).wait()

    return body


def kernel(x):
    out_c, in_c, h, w = x.shape
    hw = h * w
    xt = jnp.transpose(x, (2, 3, 0, 1)).reshape(hw, out_c, in_c)
    n = hw * out_c * in_c
    xf = xt.reshape(hw * out_c, in_c)

    b1 = 2
    maxabs = pl.pallas_call(
        _maxabs_body,
        grid=(hw // b1,),
        in_specs=[pl.BlockSpec((b1, out_c, in_c), lambda i: (i, 0, 0))],
        out_specs=pl.BlockSpec(memory_space=pltpu.SMEM),
        out_shape=jax.ShapeDtypeStruct((1, 1), jnp.float32),
    )(xt)
    scale = jnp.broadcast_to(8.0 * _DELTA / jnp.tanh(maxabs[0, 0]), (16,))

    nrows = hw * out_c
    crows = 56
    assert nrows % (crows * _NW) == 0
    nchunks = nrows // (crows * _NW)
    assert nchunks % 2 == 0
    mesh = plsc.VectorSubcoreMesh(core_axis_name="c", subcore_axis_name="s")
    out = pl.kernel(
        _sc_quant_kernel(crows, in_c, nchunks),
        mesh=mesh,
        out_type=jax.ShapeDtypeStruct((nrows, in_c), jnp.float32),
        compiler_params=pltpu.CompilerParams(needs_layout_passes=False),
        scratch_types=[
            pltpu.VMEM((crows, in_c), jnp.float32),
            pltpu.VMEM((crows, in_c), jnp.float32),
            pltpu.VMEM((crows, in_c), jnp.float32),
            pltpu.VMEM((crows, in_c), jnp.float32),
            pltpu.VMEM((16,), jnp.float32),
            pltpu.SemaphoreType.DMA,
            pltpu.SemaphoreType.DMA,
            pltpu.SemaphoreType.DMA,
            pltpu.SemaphoreType.DMA,
        ],
    )(xf, scale)
    return jnp.transpose(out.reshape(hw, out_c, in_c).reshape(h, w, out_c, in_c),
                         (2, 3, 0, 1))


# speedup vs baseline: 1.7202x; 1.0675x over previous
import jax
import jax.numpy as jnp
from jax import lax
from jax.experimental import pallas as pl
from jax.experimental.pallas import tpu as pltpu
from jax.experimental.pallas import tpu_sc as plsc

_GS = 8
_NZ = 4
_DELTA = 127.0
_MAGIC8 = 100663296.0
_INV1016 = 1.0 / (8.0 * _DELTA)

_NC, _NS = 2, 16
_NW = _NC * _NS


def _maxabs_body(x_ref, o_ref):
    i = pl.program_id(0)
    m = jnp.max(jnp.abs(x_ref[...]))

    @pl.when(i == 0)
    def _init():
        o_ref[0, 0] = m

    @pl.when(i > 0)
    def _acc():
        o_ref[0, 0] = jnp.maximum(o_ref[0, 0], m)


def _ce(a, b):
    return jnp.minimum(a, b), jnp.maximum(a, b)


def _sc_quant_unit(ibuf, obuf, sv8, sv16, rvec, cbase):
    iota8 = lax.iota(jnp.int32, 16) * _GS + cbase
    x = []
    for k in range(_GS):
        x.append(plsc.load_gather(ibuf, [rvec, iota8 + k]))
    key = []
    for k in range(_GS):
        ax = jnp.abs(x[k])
        e = jnp.exp(ax * -2.0)
        m8 = sv16 / (1.0 + e) - sv8
        key.append(((m8 + _MAGIC8) - _MAGIC8) + float(k))
    x0, x1, x2, x3, y0, y1, y2, y3 = key
    x0, x1 = _ce(x0, x1); x2, x3 = _ce(x2, x3)
    x0, x2 = _ce(x0, x2); x1, x3 = _ce(x1, x3); x1, x2 = _ce(x1, x2)
    y0, y1 = _ce(y0, y1); y2, y3 = _ce(y2, y3)
    y0, y2 = _ce(y0, y2); y1, y3 = _ce(y1, y3); y1, y2 = _ce(y1, y2)
    t = jnp.minimum(
        jnp.minimum(jnp.maximum(x0, y3), jnp.maximum(x1, y2)),
        jnp.minimum(jnp.maximum(x2, y1), jnp.maximum(x3, y0)),
    )
    for k in range(_GS):
        sc = jnp.where(x[k] < 0.0, -_INV1016, _INV1016)
        out = jnp.where(key[k] >= t, key[k] - float(k), 0.0) * sc
        plsc.store_scatter(obuf, [rvec, iota8 + k], out)


def _sc_quant_kernel(crows, ncols, nchunks):
    npairs = nchunks // 2
    cunits = ncols // 128

    def body(x_hbm, s_hbm, o_hbm, ib0, ib1, ob0, ob1, svec,
             is0, is1, os0, os1):
        wid = lax.axis_index("s") * _NC + lax.axis_index("c")
        ibufs, obufs = (ib0, ib1), (ob0, ob1)
        isems, osems = (is0, is1), (os0, os1)
        pltpu.sync_copy(s_hbm, svec)
        sv = svec[...]
        sv16 = sv + sv
        base = wid * nchunks
        zero16 = jnp.zeros((16,), jnp.int32)

        def in_copy(g, b):
            return pltpu.make_async_copy(
                x_hbm.at[pl.ds((base + g) * crows, crows)], ibufs[b],
                isems[b])

        def out_copy(g, b):
            return pltpu.make_async_copy(
                obufs[b], o_hbm.at[pl.ds((base + g) * crows, crows)],
                osems[b])

        for b in range(2):
            in_copy(b, b).start()

        def do_pair(p, _):
            for b in range(2):
                g = p * 2 + b
                in_copy(g, b).wait()

                @pl.when(p >= 1)
                def _drain():
                    out_copy(g - 2, b).wait()

                @plsc.parallel_loop(0, crows, 1, unroll=1)
                def do_row(r):
                    rvec = zero16 + r
                    for c in range(cunits):
                        _sc_quant_unit(ibufs[b], obufs[b], sv, sv16,
                                       rvec, c * 128)
                out_copy(g, b).start()

                @pl.when(p < npairs - 1)
                def _prefetch():
                    in_copy(g + 2, b).start()

            return 0

        lax.fori_loop(0, npairs, do_pair, 0)
        for b in range(2):
            out_copy(nchunks - 2 + b, b).wait()

    return body


def kernel(x):
    out_c, in_c, h, w = x.shape
    hw = h * w
    xt = jnp.transpose(x, (2, 3, 0, 1)).reshape(hw, out_c, in_c)
    n = hw * out_c * in_c
    xf = xt.reshape(hw * out_c, in_c)

    b1 = 4
    maxabs = pl.pallas_call(
        _maxabs_body,
        grid=(hw // b1,),
        in_specs=[pl.BlockSpec((b1, out_c, in_c), lambda i: (i, 0, 0))],
        out_specs=pl.BlockSpec(memory_space=pltpu.SMEM),
        out_shape=jax.ShapeDtypeStruct((1, 1), jnp.float32),
    )(xt)
    scale = jnp.broadcast_to(8.0 * _DELTA / jnp.tanh(maxabs[0, 0]), (16,))

    nrows = hw * out_c
    crows = 56
    assert nrows % (crows * _NW) == 0
    nchunks = nrows // (crows * _NW)
    assert nchunks % 2 == 0
    mesh = plsc.VectorSubcoreMesh(core_axis_name="c", subcore_axis_name="s")
    out = pl.kernel(
        _sc_quant_kernel(crows, in_c, nchunks),
        mesh=mesh,
        out_type=jax.ShapeDtypeStruct((nrows, in_c), jnp.float32),
        compiler_params=pltpu.CompilerParams(needs_layout_passes=False),
        scratch_types=[
            pltpu.VMEM((crows, in_c), jnp.float32),
            pltpu.VMEM((crows, in_c), jnp.float32),
            pltpu.VMEM((crows, in_c), jnp.float32),
            pltpu.VMEM((crows, in_c), jnp.float32),
            pltpu.VMEM((16,), jnp.float32),
            pltpu.SemaphoreType.DMA,
            pltpu.SemaphoreType.DMA,
            pltpu.SemaphoreType.DMA,
            pltpu.SemaphoreType.DMA,
        ],
    )(xf, scale)
    return jnp.transpose(out.reshape(hw, out_c, in_c).reshape(h, w, out_c, in_c),
                         (2, 3, 0, 1))
